# Initial kernel scaffold; baseline (speedup 1.0000x reference)
import jax
import jax.numpy as jnp
from jax.experimental import pallas as pl


def kernel(x, edge_index, W1, b1, W2, b2):
    raise NotImplementedError("write your pallas kernel here")


# speedup vs baseline: 34.8511x; 34.8511x over previous
import functools

import jax
import jax.numpy as jnp
from jax import lax
from jax.experimental import pallas as pl
from jax.experimental.pallas import tpu as pltpu
from jax.experimental.pallas import tpu_sc as plsc

_SC_PARAMS = pltpu.CompilerParams(use_tc_tiling_on_sc=False)

NC = 2
NS = 16
NW = NC * NS
L = 16
CHUNK = 128


def _sc_hist(dst_r, zeros_np, ones_chunk, n_pad, n_chunks):
  mesh = plsc.VectorSubcoreMesh(core_axis_name="c", subcore_axis_name="s")
  rpt = n_pad // NS

  @functools.partial(
      pl.kernel,
      out_type=jax.ShapeDtypeStruct((NC, n_pad, L), jnp.float32),
      mesh=mesh,
      compiler_params=_SC_PARAMS,
      scratch_types=[
          pltpu.VMEM((n_chunks, CHUNK), jnp.int32),
          pltpu.VMEM((CHUNK, L), jnp.float32),
          pltpu.VMEM_SHARED((n_pad, L), jnp.float32),
          pltpu.SemaphoreType.DMA,
      ],
  )
  def hist(dst_hbm, zeros_hbm, ones_hbm, out_hbm, dst_v, ones_v, acc, sem):
    c = lax.axis_index("c")
    s = lax.axis_index("s")
    wid = c * NS + s
    rows = pl.ds(s * rpt, rpt)
    pltpu.sync_copy(dst_hbm.at[wid], dst_v)
    pltpu.sync_copy(ones_hbm, ones_v)
    pltpu.sync_copy(zeros_hbm.at[rows], acc.at[rows])
    plsc.subcore_barrier()

    @pl.loop(0, n_chunks)
    def _(j):
      pltpu.sync_copy(ones_v, acc.at[dst_v.at[j]], add=True)

    plsc.subcore_barrier()
    pltpu.sync_copy(acc.at[rows], out_hbm.at[c, rows])

  return hist(dst_r, zeros_np, ones_chunk)


def _sc_edge_pass(table, src_r, dst_r, zeros_np, n_pad, n_chunks):
  mesh = plsc.VectorSubcoreMesh(core_axis_name="c", subcore_axis_name="s")
  rpt = n_pad // NS
  half = n_chunks // 2

  @functools.partial(
      pl.kernel,
      out_type=jax.ShapeDtypeStruct((NC, n_pad, L), jnp.float32),
      mesh=mesh,
      compiler_params=_SC_PARAMS,
      scratch_types=[
          pltpu.VMEM((n_chunks, CHUNK), jnp.int32),
          pltpu.VMEM((n_chunks, CHUNK), jnp.int32),
          pltpu.VMEM((CHUNK, L), jnp.float32),
          pltpu.VMEM((CHUNK, L), jnp.float32),
          pltpu.VMEM_SHARED((n_pad, L), jnp.float32),
          pltpu.SemaphoreType.DMA,
          pltpu.SemaphoreType.DMA,
      ],
  )
  def edge_pass(tab_hbm, src_hbm, dst_hbm, zeros_hbm, out_hbm,
                src_v, dst_v, buf0, buf1, acc, sem0, sem1):
    c = lax.axis_index("c")
    s = lax.axis_index("s")
    wid = c * NS + s
    rows = pl.ds(s * rpt, rpt)
    pltpu.sync_copy(src_hbm.at[wid], src_v)
    pltpu.sync_copy(dst_hbm.at[wid], dst_v)
    pltpu.sync_copy(zeros_hbm.at[rows], acc.at[rows])
    plsc.subcore_barrier()

    pltpu.async_copy(tab_hbm.at[src_v.at[0]], buf0, sem0)
    pltpu.async_copy(tab_hbm.at[src_v.at[1]], buf1, sem1)

    @pl.loop(0, half)
    def _(t):
      j0 = 2 * t
      j1 = j0 + 1
      pltpu.make_async_copy(tab_hbm.at[src_v.at[j0]], buf0, sem0).wait()
      pltpu.sync_copy(buf0, acc.at[dst_v.at[j0]], add=True)

      @pl.when(t + 1 < half)
      def _():
        pltpu.async_copy(tab_hbm.at[src_v.at[j0 + 2]], buf0, sem0)

      pltpu.make_async_copy(tab_hbm.at[src_v.at[j1]], buf1, sem1).wait()
      pltpu.sync_copy(buf1, acc.at[dst_v.at[j1]], add=True)

      @pl.when(t + 1 < half)
      def _():
        pltpu.async_copy(tab_hbm.at[src_v.at[j1 + 2]], buf1, sem1)

    plsc.subcore_barrier()
    pltpu.sync_copy(acc.at[rows], out_hbm.at[c, rows])

  return edge_pass(table, src_r, dst_r, zeros_np)


def _tc_prep(x, w1, c0, c1, blk):
  n, d_in = x.shape
  d_hid = w1.shape[1]
  grid = (n // blk,)

  def body(x_ref, w1_ref, c0_ref, c1_ref, dis_ref, xws_ref):
    deg = c0_ref[...] + c1_ref[...] + 1.0
    dis = lax.rsqrt(deg)
    xw = jnp.dot(x_ref[...], w1_ref[...], preferred_element_type=jnp.float32)
    dis_ref[...] = dis
    xws_ref[...] = xw * dis

  return pl.pallas_call(
      body,
      grid=grid,
      in_specs=[
          pl.BlockSpec((blk, d_in), lambda i: (i, 0)),
          pl.BlockSpec((d_in, d_hid), lambda i: (0, 0)),
          pl.BlockSpec((blk, L), lambda i: (i, 0)),
          pl.BlockSpec((blk, L), lambda i: (i, 0)),
      ],
      out_specs=[
          pl.BlockSpec((blk, L), lambda i: (i, 0)),
          pl.BlockSpec((blk, d_hid), lambda i: (i, 0)),
      ],
      out_shape=[
          jax.ShapeDtypeStruct((n, L), jnp.float32),
          jax.ShapeDtypeStruct((n, d_hid), jnp.float32),
      ],
  )(x, w1, c0, c1)


def _tc_mid(s0, s1, xws1, dis, b1, w2, blk):
  n, d_hid = xws1.shape
  d_out = w2.shape[1]
  grid = (n // blk,)

  def body(s0_ref, s1_ref, xws_ref, dis_ref, b1_ref, w2_ref, out_ref):
    h = (s0_ref[...] + s1_ref[...] + xws_ref[...]) * dis_ref[...] + b1_ref[...]
    h = jnp.maximum(h, 0.0)
    xw2 = jnp.dot(h, w2_ref[...], preferred_element_type=jnp.float32)
    out_ref[...] = xw2 * dis_ref[...]

  return pl.pallas_call(
      body,
      grid=grid,
      in_specs=[
          pl.BlockSpec((blk, d_hid), lambda i: (i, 0)),
          pl.BlockSpec((blk, d_hid), lambda i: (i, 0)),
          pl.BlockSpec((blk, d_hid), lambda i: (i, 0)),
          pl.BlockSpec((blk, L), lambda i: (i, 0)),
          pl.BlockSpec((1, d_hid), lambda i: (0, 0)),
          pl.BlockSpec((d_hid, d_out), lambda i: (0, 0)),
      ],
      out_specs=pl.BlockSpec((blk, d_out), lambda i: (i, 0)),
      out_shape=jax.ShapeDtypeStruct((n, d_out), jnp.float32),
  )(s0, s1, xws1, dis, b1, w2)


def _tc_fin(s0, s1, xws2, dis, b2, blk):
  n, d_out = xws2.shape
  grid = (n // blk,)

  def body(s0_ref, s1_ref, xws_ref, dis_ref, b2_ref, out_ref):
    out_ref[...] = ((s0_ref[...] + s1_ref[...] + xws_ref[...]) * dis_ref[...]
                    + b2_ref[...])

  return pl.pallas_call(
      body,
      grid=grid,
      in_specs=[
          pl.BlockSpec((blk, d_out), lambda i: (i, 0)),
          pl.BlockSpec((blk, d_out), lambda i: (i, 0)),
          pl.BlockSpec((blk, d_out), lambda i: (i, 0)),
          pl.BlockSpec((blk, L), lambda i: (i, 0)),
          pl.BlockSpec((1, d_out), lambda i: (0, 0)),
      ],
      out_specs=pl.BlockSpec((blk, d_out), lambda i: (i, 0)),
      out_shape=jax.ShapeDtypeStruct((n, d_out), jnp.float32),
  )(s0, s1, xws2, dis, b2)


def kernel(x, edge_index, W1, b1, W2, b2):
  n = x.shape[0]
  e = edge_index.shape[1]

  n_chunks = -(-e // (NW * CHUNK))
  if n_chunks % 2:
    n_chunks += 1
  e_pad = NW * n_chunks * CHUNK
  n_pad = -(-(n + 1) // (NS * 8)) * (NS * 8)

  src = edge_index[0]
  dst = edge_index[1]
  pad = e_pad - e
  src_r = jnp.concatenate(
      [src, jnp.zeros((pad,), jnp.int32)]).reshape(NW, n_chunks, CHUNK)
  dst_r = jnp.concatenate(
      [dst, jnp.full((pad,), n, jnp.int32)]).reshape(NW, n_chunks, CHUNK)
  zeros_np = jnp.zeros((n_pad, L), jnp.float32)
  ones_chunk = jnp.ones((CHUNK, L), jnp.float32)

  cpart = _sc_hist(dst_r, zeros_np, ones_chunk, n_pad, n_chunks)
  dis, xws1 = _tc_prep(x, W1, cpart[0, :n], cpart[1, :n], blk=1000)

  s1 = _sc_edge_pass(xws1, src_r, dst_r, zeros_np, n_pad, n_chunks)
  xws2 = _tc_mid(s1[0, :n], s1[1, :n], xws1, dis,
                 b1.reshape(1, -1), W2, blk=1000)

  s2 = _sc_edge_pass(xws2, src_r, dst_r, zeros_np, n_pad, n_chunks)
  out = _tc_fin(s2[0, :n], s2[1, :n], xws2, dis, b2.reshape(1, -1), blk=1000)
  return out


# speedup vs baseline: 52.2041x; 1.4979x over previous
import functools

import jax
import jax.numpy as jnp
from jax import lax
from jax.experimental import pallas as pl
from jax.experimental.pallas import tpu as pltpu
from jax.experimental.pallas import tpu_sc as plsc

_SC_PARAMS = pltpu.CompilerParams(use_tc_tiling_on_sc=False)

NC = 2
NS = 16
NW = NC * NS
L = 16
CHUNK = 128


def _sc_hist(dst_r, zeros_np, ones_chunk, n_pad, n_chunks):
  mesh = plsc.VectorSubcoreMesh(core_axis_name="c", subcore_axis_name="s")
  rpt = n_pad // NS

  @functools.partial(
      pl.kernel,
      out_type=jax.ShapeDtypeStruct((NC, n_pad, L), jnp.float32),
      mesh=mesh,
      compiler_params=_SC_PARAMS,
      scratch_types=[
          pltpu.VMEM((n_chunks, CHUNK), jnp.int32),
          pltpu.VMEM((CHUNK, L), jnp.float32),
          pltpu.VMEM_SHARED((n_pad, L), jnp.float32),
          pltpu.SemaphoreType.DMA,
      ],
  )
  def hist(dst_hbm, zeros_hbm, ones_hbm, out_hbm, dst_v, ones_v, acc, sem):
    c = lax.axis_index("c")
    s = lax.axis_index("s")
    wid = c * NS + s
    rows = pl.ds(s * rpt, rpt)
    pltpu.sync_copy(dst_hbm.at[wid], dst_v)
    pltpu.sync_copy(ones_hbm, ones_v)
    pltpu.sync_copy(zeros_hbm.at[rows], acc.at[rows])
    plsc.subcore_barrier()

    @pl.loop(0, n_chunks)
    def _(j):
      pltpu.async_copy(ones_v, acc.at[dst_v.at[j]], sem, add=True)

    @pl.loop(0, n_chunks)
    def _(j):
      pltpu.make_async_copy(ones_v, acc.at[dst_v.at[j]], sem).wait()

    plsc.subcore_barrier()
    pltpu.sync_copy(acc.at[rows], out_hbm.at[c, rows])

  return hist(dst_r, zeros_np, ones_chunk)


def _sc_edge_pass(table, src_r, dst_r, zeros_np, n_pad, n_chunks):
  mesh = plsc.VectorSubcoreMesh(core_axis_name="c", subcore_axis_name="s")
  rpt = n_pad // NS
  nbuf = 8
  outer = n_chunks // nbuf

  @functools.partial(
      pl.kernel,
      out_type=jax.ShapeDtypeStruct((NC, n_pad, L), jnp.float32),
      mesh=mesh,
      compiler_params=_SC_PARAMS,
      scratch_types=[
          pltpu.VMEM((n_chunks, CHUNK), jnp.int32),
          pltpu.VMEM((n_chunks, CHUNK), jnp.int32),
          pltpu.VMEM((nbuf, CHUNK, L), jnp.float32),
          pltpu.VMEM_SHARED((n_pad, L), jnp.float32),
          pltpu.SemaphoreType.DMA((nbuf,)),
          pltpu.SemaphoreType.DMA((nbuf,)),
      ],
  )
  def edge_pass(tab_hbm, src_hbm, dst_hbm, zeros_hbm, out_hbm,
                src_v, dst_v, bufs, acc, gsem, ssem):
    c = lax.axis_index("c")
    s = lax.axis_index("s")
    wid = c * NS + s
    rows = pl.ds(s * rpt, rpt)
    pltpu.sync_copy(src_hbm.at[wid], src_v)
    pltpu.sync_copy(dst_hbm.at[wid], dst_v)
    pltpu.sync_copy(zeros_hbm.at[rows], acc.at[rows])
    plsc.subcore_barrier()

    for b in range(nbuf):
      pltpu.async_copy(tab_hbm.at[src_v.at[b]], bufs.at[b], gsem.at[b])

    @pl.loop(0, outer)
    def _(t):
      base = t * nbuf
      for b in range(nbuf):
        j = base + b
        pltpu.make_async_copy(
            tab_hbm.at[src_v.at[j]], bufs.at[b], gsem.at[b]).wait()
        pltpu.async_copy(bufs.at[b], acc.at[dst_v.at[j]], ssem.at[b],
                         add=True)
      for b in range(nbuf):
        j = base + b
        pltpu.make_async_copy(
            bufs.at[b], acc.at[dst_v.at[j]], ssem.at[b]).wait()

        @pl.when(t + 1 < outer)
        def _():
          pltpu.async_copy(
              tab_hbm.at[src_v.at[j + nbuf]], bufs.at[b], gsem.at[b])

    plsc.subcore_barrier()
    pltpu.sync_copy(acc.at[rows], out_hbm.at[c, rows])

  return edge_pass(table, src_r, dst_r, zeros_np)


def _tc_prep(x, w1, c0, c1, blk):
  n, d_in = x.shape
  d_hid = w1.shape[1]
  grid = (n // blk,)

  def body(x_ref, w1_ref, c0_ref, c1_ref, dis_ref, xws_ref):
    deg = c0_ref[...] + c1_ref[...] + 1.0
    dis = lax.rsqrt(deg)
    xw = jnp.dot(x_ref[...], w1_ref[...], preferred_element_type=jnp.float32)
    dis_ref[...] = dis
    xws_ref[...] = xw * dis

  return pl.pallas_call(
      body,
      grid=grid,
      in_specs=[
          pl.BlockSpec((blk, d_in), lambda i: (i, 0)),
          pl.BlockSpec((d_in, d_hid), lambda i: (0, 0)),
          pl.BlockSpec((blk, L), lambda i: (i, 0)),
          pl.BlockSpec((blk, L), lambda i: (i, 0)),
      ],
      out_specs=[
          pl.BlockSpec((blk, L), lambda i: (i, 0)),
          pl.BlockSpec((blk, d_hid), lambda i: (i, 0)),
      ],
      out_shape=[
          jax.ShapeDtypeStruct((n, L), jnp.float32),
          jax.ShapeDtypeStruct((n, d_hid), jnp.float32),
      ],
  )(x, w1, c0, c1)


def _tc_mid(s0, s1, xws1, dis, b1, w2, blk):
  n, d_hid = xws1.shape
  d_out = w2.shape[1]
  grid = (n // blk,)

  def body(s0_ref, s1_ref, xws_ref, dis_ref, b1_ref, w2_ref, out_ref):
    h = (s0_ref[...] + s1_ref[...] + xws_ref[...]) * dis_ref[...] + b1_ref[...]
    h = jnp.maximum(h, 0.0)
    xw2 = jnp.dot(h, w2_ref[...], preferred_element_type=jnp.float32)
    out_ref[...] = xw2 * dis_ref[...]

  return pl.pallas_call(
      body,
      grid=grid,
      in_specs=[
          pl.BlockSpec((blk, d_hid), lambda i: (i, 0)),
          pl.BlockSpec((blk, d_hid), lambda i: (i, 0)),
          pl.BlockSpec((blk, d_hid), lambda i: (i, 0)),
          pl.BlockSpec((blk, L), lambda i: (i, 0)),
          pl.BlockSpec((1, d_hid), lambda i: (0, 0)),
          pl.BlockSpec((d_hid, d_out), lambda i: (0, 0)),
      ],
      out_specs=pl.BlockSpec((blk, d_out), lambda i: (i, 0)),
      out_shape=jax.ShapeDtypeStruct((n, d_out), jnp.float32),
  )(s0, s1, xws1, dis, b1, w2)


def _tc_fin(s0, s1, xws2, dis, b2, blk):
  n, d_out = xws2.shape
  grid = (n // blk,)

  def body(s0_ref, s1_ref, xws_ref, dis_ref, b2_ref, out_ref):
    out_ref[...] = ((s0_ref[...] + s1_ref[...] + xws_ref[...]) * dis_ref[...]
                    + b2_ref[...])

  return pl.pallas_call(
      body,
      grid=grid,
      in_specs=[
          pl.BlockSpec((blk, d_out), lambda i: (i, 0)),
          pl.BlockSpec((blk, d_out), lambda i: (i, 0)),
          pl.BlockSpec((blk, d_out), lambda i: (i, 0)),
          pl.BlockSpec((blk, L), lambda i: (i, 0)),
          pl.BlockSpec((1, d_out), lambda i: (0, 0)),
      ],
      out_specs=pl.BlockSpec((blk, d_out), lambda i: (i, 0)),
      out_shape=jax.ShapeDtypeStruct((n, d_out), jnp.float32),
  )(s0, s1, xws2, dis, b2)


def kernel(x, edge_index, W1, b1, W2, b2):
  n = x.shape[0]
  e = edge_index.shape[1]

  n_chunks = -(-e // (NW * CHUNK))
  n_chunks = -(-n_chunks // 8) * 8
  e_pad = NW * n_chunks * CHUNK
  n_pad = -(-(n + 1) // (NS * 8)) * (NS * 8)

  src = edge_index[0]
  dst = edge_index[1]
  pad = e_pad - e
  pad_idx = jnp.arange(pad, dtype=jnp.int32)
  src_r = jnp.concatenate(
      [src, pad_idx % n]).reshape(NW, n_chunks, CHUNK)
  dst_r = jnp.concatenate(
      [dst, n + pad_idx % (n_pad - n)]).reshape(NW, n_chunks, CHUNK)
  zeros_np = jnp.zeros((n_pad, L), jnp.float32)
  ones_chunk = jnp.ones((CHUNK, L), jnp.float32)

  cpart = _sc_hist(dst_r, zeros_np, ones_chunk, n_pad, n_chunks)
  dis, xws1 = _tc_prep(x, W1, cpart[0, :n], cpart[1, :n], blk=1000)

  s1 = _sc_edge_pass(xws1, src_r, dst_r, zeros_np, n_pad, n_chunks)
  xws2 = _tc_mid(s1[0, :n], s1[1, :n], xws1, dis,
                 b1.reshape(1, -1), W2, blk=1000)

  s2 = _sc_edge_pass(xws2, src_r, dst_r, zeros_np, n_pad, n_chunks)
  out = _tc_fin(s2[0, :n], s2[1, :n], xws2, dis, b2.reshape(1, -1), blk=1000)
  return out
